# baseline (speedup 1.0000x reference)
import functools

import jax
import jax.numpy as jnp
from jax import lax
from jax.experimental import pallas as pl
from jax.experimental.pallas import tpu as pltpu
from jax.experimental.pallas import tpu_sc as plsc

MAX_REL = 128
EMBED_DIM = 32
SEQ_LEN = 2048

_NC = 2
_NS = 16
_NW = _NC * _NS
_R = SEQ_LEN // _NW
_U = 2560
_DB = EMBED_DIM // 8
_JT = SEQ_LEN // 128


import numpy as _np

_COL16 = _np.concatenate(
    [
        _np.clip(_np.arange(288)[None, :] - _np.arange(16)[:, None], 0, 256),
        _np.zeros((16, 16), _np.int64),
        _np.full((16, 16), 256, _np.int64),
    ],
    axis=1,
)
_FLAT16 = (
    (256 - _COL16[:, None, :]) * EMBED_DIM + _np.arange(EMBED_DIM)[None, :, None]
).astype(_np.int32)


def _phase_tables(weight):
    return weight.reshape(-1)[_FLAT16]


def kernel(inputs, weight):
    del inputs

    mesh = plsc.VectorSubcoreMesh(core_axis_name="c", subcore_axis_name="s")

    @functools.partial(
        pl.kernel,
        mesh=mesh,
        compiler_params=pltpu.CompilerParams(use_tc_tiling_on_sc=False),
        out_type=jax.ShapeDtypeStruct((SEQ_LEN, _DB, _JT, 8, 128), jnp.float32),
        scratch_types=[
            pltpu.VMEM((EMBED_DIM, 320), jnp.float32),
            pltpu.VMEM((_DB, 8, _U), jnp.float32),
            pltpu.SemaphoreType.DMA,
        ],
    )
    def band_embed(wrt16_hbm, out_hbm, wv, pv_v, sem):
        wid = lax.axis_index("s") * _NC + lax.axis_index("c")
        a = lax.rem(wid, 8)
        g = lax.div(wid, 8)
        first = a + 512 * g
        cb = first + 504
        ph = lax.rem(cb - 128, 16)
        base = pl.multiple_of(cb - 128 - ph, 16)

        pltpu.sync_copy(wrt16_hbm.at[ph], wv)

        nl = lax.div(base, 16)
        nl4 = lax.div(nl, 4)
        nr0 = nl + 18
        nr4 = lax.div(_U // 16 - nr0, 4)
        for d in range(EMBED_DIM):
            rb, dd = d // 8, d % 8
            leftv = wv[d, pl.ds(288, 16)]
            rightv = wv[d, pl.ds(304, 16)]

            def lfill4(n, carry, leftv=leftv, rb=rb, dd=dd):
                for m in range(4):
                    pv_v[rb, dd, pl.ds((n * 4 + m) * 16, 16)] = leftv
                return carry

            lax.fori_loop(0, nl4, lfill4, 0)

            def lfill(n, carry, leftv=leftv, rb=rb, dd=dd):
                pv_v[rb, dd, pl.ds(n * 16, 16)] = leftv
                return carry

            lax.fori_loop(nl4 * 4, nl, lfill, 0)

            def rfill4(n, carry, rightv=rightv, rb=rb, dd=dd):
                for m in range(4):
                    pv_v[rb, dd, pl.ds((nr0 + n * 4 + m) * 16, 16)] = rightv
                return carry

            lax.fori_loop(0, nr4, rfill4, 0)

            def rfill(n, carry, rightv=rightv, rb=rb, dd=dd):
                pv_v[rb, dd, pl.ds(n * 16, 16)] = rightv
                return carry

            lax.fori_loop(nr0 + nr4 * 4, _U // 16, rfill, 0)
            for mm in range(18):
                pv_v[rb, dd, pl.ds(base + 16 * mm, 16)] = wv[d, pl.ds(16 * mm, 16)]

        def plane(k, carry):
            o = pl.multiple_of(504 - 8 * k, 8)
            for C in range(_JT):
                pltpu.make_async_copy(
                    pv_v.at[:, :, pl.ds(o + 128 * C, 128)],
                    out_hbm.at[first + 8 * k, :, C],
                    sem,
                ).start()
            return carry

        lax.fori_loop(0, _R, plane, 0)

        def drain(k, carry):
            for C in range(_JT):
                pltpu.make_async_copy(
                    pv_v.at[:, :, pl.ds(128 * C, 128)],
                    out_hbm.at[first, :, C],
                    sem,
                ).wait()
            return carry

        lax.fori_loop(0, _R, drain, 0)

    o5 = band_embed(_phase_tables(weight))
    out_t = jnp.transpose(o5, (0, 1, 3, 2, 4)).reshape(SEQ_LEN, EMBED_DIM, SEQ_LEN)
    return jnp.transpose(out_t, (0, 2, 1))


# speedup vs baseline: 10.2929x; 10.2929x over previous
import functools

import jax
import jax.numpy as jnp
from jax import lax
from jax.experimental import pallas as pl
from jax.experimental.pallas import tpu as pltpu
from jax.experimental.pallas import tpu_sc as plsc

MAX_REL = 128
EMBED_DIM = 32
SEQ_LEN = 2048

_NC = 2
_NS = 16
_NW = _NC * _NS
_R = SEQ_LEN // _NW
_U = 2560
_DB = EMBED_DIM // 8
_JT = SEQ_LEN // 128


import numpy as _np

def _phase_tables(weight):
    wrt = jnp.flip(weight, axis=0).T
    first = wrt[:, :1]
    last = wrt[:, 256:]
    sh = jnp.stack([
        jnp.concatenate(
            [jnp.repeat(first, p, axis=1), wrt, jnp.repeat(last, 31 - p, axis=1)],
            axis=1,
        )
        for p in range(16)
    ])
    consts = jnp.concatenate(
        [jnp.repeat(first, 16, axis=1), jnp.repeat(last, 16, axis=1)], axis=1
    )
    return jnp.concatenate(
        [sh, jnp.broadcast_to(consts[None], (16, 32, 32))], axis=2
    )


def kernel(inputs, weight):
    del inputs

    mesh = plsc.VectorSubcoreMesh(core_axis_name="c", subcore_axis_name="s")

    @functools.partial(
        pl.kernel,
        mesh=mesh,
        compiler_params=pltpu.CompilerParams(use_tc_tiling_on_sc=False),
        out_type=jax.ShapeDtypeStruct((SEQ_LEN, _DB, _JT, 8, 128), jnp.float32),
        scratch_types=[
            pltpu.VMEM((EMBED_DIM, 320), jnp.float32),
            pltpu.VMEM((_DB, 8, _U), jnp.float32),
            pltpu.SemaphoreType.DMA,
        ],
    )
    def band_embed(wrt16_hbm, out_hbm, wv, pv_v, sem):
        wid = lax.axis_index("s") * _NC + lax.axis_index("c")
        a = lax.rem(wid, 8)
        g = lax.div(wid, 8)
        first = a + 512 * g
        cb = first + 504
        ph = lax.rem(cb - 128, 16)
        base = pl.multiple_of(cb - 128 - ph, 16)

        pltpu.sync_copy(wrt16_hbm.at[ph], wv)

        nl = lax.div(base, 16)
        nl4 = lax.div(nl, 4)
        nr0 = nl + 18
        nr4 = lax.div(_U // 16 - nr0, 4)
        for d in range(EMBED_DIM):
            rb, dd = d // 8, d % 8
            leftv = wv[d, pl.ds(288, 16)]
            rightv = wv[d, pl.ds(304, 16)]

            def lfill4(n, carry, leftv=leftv, rb=rb, dd=dd):
                for m in range(4):
                    pv_v[rb, dd, pl.ds((n * 4 + m) * 16, 16)] = leftv
                return carry

            lax.fori_loop(0, nl4, lfill4, 0)

            def lfill(n, carry, leftv=leftv, rb=rb, dd=dd):
                pv_v[rb, dd, pl.ds(n * 16, 16)] = leftv
                return carry

            lax.fori_loop(nl4 * 4, nl, lfill, 0)

            def rfill4(n, carry, rightv=rightv, rb=rb, dd=dd):
                for m in range(4):
                    pv_v[rb, dd, pl.ds((nr0 + n * 4 + m) * 16, 16)] = rightv
                return carry

            lax.fori_loop(0, nr4, rfill4, 0)

            def rfill(n, carry, rightv=rightv, rb=rb, dd=dd):
                pv_v[rb, dd, pl.ds(n * 16, 16)] = rightv
                return carry

            lax.fori_loop(nr0 + nr4 * 4, _U // 16, rfill, 0)
            for mm in range(18):
                pv_v[rb, dd, pl.ds(base + 16 * mm, 16)] = wv[d, pl.ds(16 * mm, 16)]

        def plane(k, carry):
            o = pl.multiple_of(504 - 8 * k, 8)
            for C in range(_JT):
                pltpu.make_async_copy(
                    pv_v.at[:, :, pl.ds(o + 128 * C, 128)],
                    out_hbm.at[first + 8 * k, :, C],
                    sem,
                ).start()
            return carry

        lax.fori_loop(0, _R, plane, 0)

        def drain(k, carry):
            for C in range(_JT):
                pltpu.make_async_copy(
                    pv_v.at[:, :, pl.ds(128 * C, 128)],
                    out_hbm.at[first, :, C],
                    sem,
                ).wait()
            return carry

        lax.fori_loop(0, _R, drain, 0)

    o5 = band_embed(_phase_tables(weight))
    out_t = jnp.transpose(o5, (0, 1, 3, 2, 4)).reshape(SEQ_LEN, EMBED_DIM, SEQ_LEN)
    return jnp.transpose(out_t, (0, 2, 1))


# speedup vs baseline: 10.3067x; 1.0013x over previous
import functools

import jax
import jax.numpy as jnp
from jax import lax
from jax.experimental import pallas as pl
from jax.experimental.pallas import tpu as pltpu
from jax.experimental.pallas import tpu_sc as plsc

MAX_REL = 128
EMBED_DIM = 32
SEQ_LEN = 2048

_NC = 2
_NS = 16
_NW = _NC * _NS
_R = SEQ_LEN // _NW
_U = 2560
_DB = EMBED_DIM // 8
_JT = SEQ_LEN // 128


import numpy as _np

def _phase_tables(weight):
    wrt = jnp.flip(weight, axis=0).T
    first = wrt[:, :1]
    last = wrt[:, 256:]
    sh = jnp.stack([
        jnp.concatenate(
            [jnp.repeat(first, p, axis=1), wrt, jnp.repeat(last, 31 - p, axis=1)],
            axis=1,
        )
        for p in range(8, 16)
    ])
    consts = jnp.concatenate(
        [jnp.repeat(first, 16, axis=1), jnp.repeat(last, 16, axis=1)], axis=1
    )
    return jnp.concatenate(
        [sh, jnp.broadcast_to(consts[None], (8, 32, 32))], axis=2
    )


def kernel(inputs, weight):
    del inputs

    mesh = plsc.VectorSubcoreMesh(core_axis_name="c", subcore_axis_name="s")

    @functools.partial(
        pl.kernel,
        mesh=mesh,
        compiler_params=pltpu.CompilerParams(use_tc_tiling_on_sc=False),
        out_type=jax.ShapeDtypeStruct((SEQ_LEN, _DB, _JT, 8, 128), jnp.float32),
        scratch_types=[
            pltpu.VMEM((EMBED_DIM, 320), jnp.float32),
            pltpu.VMEM((_DB, 8, _U), jnp.float32),
            pltpu.SemaphoreType.DMA,
        ],
    )
    def band_embed(wrt16_hbm, out_hbm, wv, pv_v, sem):
        wid = lax.axis_index("s") * _NC + lax.axis_index("c")
        a = lax.rem(wid, 8)
        g = lax.div(wid, 8)
        first = a + 512 * g
        cb = first + 504
        base = pl.multiple_of(cb - 128 - (a + 8), 16)

        pltpu.sync_copy(wrt16_hbm.at[a], wv)

        nl = lax.div(base, 16)
        nl4 = lax.div(nl, 4)
        nr0 = nl + 18
        nr4 = lax.div(_U // 16 - nr0, 4)
        for d in range(EMBED_DIM):
            rb, dd = d // 8, d % 8
            leftv = wv[d, pl.ds(288, 16)]
            rightv = wv[d, pl.ds(304, 16)]

            def lfill4(n, carry, leftv=leftv, rb=rb, dd=dd):
                for m in range(4):
                    pv_v[rb, dd, pl.ds((n * 4 + m) * 16, 16)] = leftv
                return carry

            lax.fori_loop(0, nl4, lfill4, 0)

            def lfill(n, carry, leftv=leftv, rb=rb, dd=dd):
                pv_v[rb, dd, pl.ds(n * 16, 16)] = leftv
                return carry

            lax.fori_loop(nl4 * 4, nl, lfill, 0)

            def rfill4(n, carry, rightv=rightv, rb=rb, dd=dd):
                for m in range(4):
                    pv_v[rb, dd, pl.ds((nr0 + n * 4 + m) * 16, 16)] = rightv
                return carry

            lax.fori_loop(0, nr4, rfill4, 0)

            def rfill(n, carry, rightv=rightv, rb=rb, dd=dd):
                pv_v[rb, dd, pl.ds(n * 16, 16)] = rightv
                return carry

            lax.fori_loop(nr0 + nr4 * 4, _U // 16, rfill, 0)
            for mm in range(18):
                pv_v[rb, dd, pl.ds(base + 16 * mm, 16)] = wv[d, pl.ds(16 * mm, 16)]

        def plane(k, carry):
            o = pl.multiple_of(504 - 8 * k, 8)
            for C in range(_JT):
                pltpu.make_async_copy(
                    pv_v.at[:, :, pl.ds(o + 128 * C, 128)],
                    out_hbm.at[first + 8 * k, :, C],
                    sem,
                ).start()
            return carry

        lax.fori_loop(0, _R, plane, 0)

        def drain(k, carry):
            for C in range(_JT):
                pltpu.make_async_copy(
                    pv_v.at[:, :, pl.ds(128 * C, 128)],
                    out_hbm.at[first, :, C],
                    sem,
                ).wait()
            return carry

        lax.fori_loop(0, _R, drain, 0)

    o5 = band_embed(_phase_tables(weight))
    out_t = jnp.transpose(o5, (0, 1, 3, 2, 4)).reshape(SEQ_LEN, EMBED_DIM, SEQ_LEN)
    return jnp.transpose(out_t, (0, 2, 1))


# speedup vs baseline: 10.3126x; 1.0006x over previous
import functools

import jax
import jax.numpy as jnp
from jax import lax
from jax.experimental import pallas as pl
from jax.experimental.pallas import tpu as pltpu
from jax.experimental.pallas import tpu_sc as plsc

MAX_REL = 128
EMBED_DIM = 32
SEQ_LEN = 2048

_NC = 2
_NS = 16
_NW = _NC * _NS
_R = SEQ_LEN // _NW
_U = 2560
_DB = EMBED_DIM // 8
_JT = SEQ_LEN // 128


def _phase_tables(weight):
    wrt = jnp.flip(weight, axis=0).T
    first = wrt[:, :1]
    last = wrt[:, 256:]
    sh = jnp.stack([
        jnp.concatenate(
            [jnp.repeat(first, p, axis=1), wrt, jnp.repeat(last, 31 - p, axis=1)],
            axis=1,
        )
        for p in range(8, 16)
    ])
    consts = jnp.concatenate(
        [jnp.repeat(first, 16, axis=1), jnp.repeat(last, 16, axis=1)], axis=1
    )
    return jnp.concatenate(
        [sh, jnp.broadcast_to(consts[None], (8, 32, 32))], axis=2
    )


def kernel(inputs, weight):
    del inputs

    mesh = plsc.VectorSubcoreMesh(core_axis_name="c", subcore_axis_name="s")

    @functools.partial(
        pl.kernel,
        mesh=mesh,
        compiler_params=pltpu.CompilerParams(use_tc_tiling_on_sc=False),
        out_type=jax.ShapeDtypeStruct((SEQ_LEN, _DB, _JT, 8, 128), jnp.float32),
        scratch_types=[
            pltpu.VMEM((EMBED_DIM, 320), jnp.float32),
            pltpu.VMEM((_DB, 8, _U), jnp.float32),
            pltpu.SemaphoreType.DMA,
        ],
    )
    def band_embed(wrt16_hbm, out_hbm, wv, pv_v, sem):
        wid = lax.axis_index("s") * _NC + lax.axis_index("c")
        a = lax.rem(wid, 8)
        g = lax.div(wid, 8)
        first = a + 512 * g
        cb = first + 504
        base = pl.multiple_of(cb - 128 - (a + 8), 16)

        pltpu.sync_copy(wrt16_hbm.at[a], wv)

        nl = lax.div(base, 16)
        nl4 = lax.div(nl, 4)
        nr0 = nl + 18
        nr4 = lax.div(_U // 16 - nr0, 4)
        for d in range(EMBED_DIM):
            rb, dd = d // 8, d % 8
            leftv = wv[d, pl.ds(288, 16)]
            rightv = wv[d, pl.ds(304, 16)]

            def lfill4(n, carry, leftv=leftv, rb=rb, dd=dd):
                for m in range(4):
                    pv_v[rb, dd, pl.ds((n * 4 + m) * 16, 16)] = leftv
                return carry

            lax.fori_loop(0, nl4, lfill4, 0)

            def lfill(n, carry, leftv=leftv, rb=rb, dd=dd):
                pv_v[rb, dd, pl.ds(n * 16, 16)] = leftv
                return carry

            lax.fori_loop(nl4 * 4, nl, lfill, 0)

            def rfill4(n, carry, rightv=rightv, rb=rb, dd=dd):
                for m in range(4):
                    pv_v[rb, dd, pl.ds((nr0 + n * 4 + m) * 16, 16)] = rightv
                return carry

            lax.fori_loop(0, nr4, rfill4, 0)

            def rfill(n, carry, rightv=rightv, rb=rb, dd=dd):
                pv_v[rb, dd, pl.ds(n * 16, 16)] = rightv
                return carry

            lax.fori_loop(nr0 + nr4 * 4, _U // 16, rfill, 0)
            for mm in range(18):
                pv_v[rb, dd, pl.ds(base + 16 * mm, 16)] = wv[d, pl.ds(16 * mm, 16)]

        def plane(k, carry):
            o = pl.multiple_of(504 - 8 * k, 8)
            for C in range(_JT):
                pltpu.make_async_copy(
                    pv_v.at[:, :, pl.ds(o + 128 * C, 128)],
                    out_hbm.at[first + 8 * k, :, C],
                    sem,
                ).start()
            return carry

        lax.fori_loop(0, _R, plane, 0)

        def drain(k, carry):
            for C in range(_JT):
                pltpu.make_async_copy(
                    pv_v.at[:, :, pl.ds(128 * C, 128)],
                    out_hbm.at[first, :, C],
                    sem,
                ).wait()
            return carry

        lax.fori_loop(0, _R, drain, 0)

    o5 = band_embed(_phase_tables(weight))
    out_t = jnp.transpose(o5, (0, 1, 3, 2, 4)).reshape(SEQ_LEN, EMBED_DIM, SEQ_LEN)
    return jnp.transpose(out_t, (0, 2, 1))
